# baseline (speedup 1.0000x reference)
import functools

import jax
import jax.numpy as jnp
from jax import lax
from jax.experimental import pallas as pl
from jax.experimental.pallas import tpu as pltpu
from jax.experimental.pallas import tpu_sc as plsc

N_LAYERS = 4
C = 1.0
N, E, D, H = 10000, 320000, 128, 128

NC, NS = 2, 16
NW = NC * NS
EC = 128
CPW = 80
NCHUNK_PAD = NW * CPW
E_PAD = NCHUNK_PAD * EC
VPL = H // 16

NBG = 2
NGRP_G = CPW // NBG
NBS = 2
NGRP_S = CPW // NBS

NP = 10240
ROWS_PER_TILE = NP // NS
ZR = 128

_mesh = plsc.VectorSubcoreMesh(core_axis_name="c", subcore_axis_name="s")


def _wid():
    return lax.axis_index("s") * NC + lax.axis_index("c")


@functools.partial(
    pl.kernel,
    out_type=jax.ShapeDtypeStruct((E_PAD, H), jnp.float32),
    mesh=_mesh,
    scratch_types=[
        pltpu.VMEM((EC,), jnp.int32),
        pltpu.VMEM((EC,), jnp.int32),
        pltpu.VMEM((EC, H), jnp.float32),
        pltpu.VMEM((EC, H), jnp.float32),
        pltpu.SemaphoreType.DMA,
        pltpu.SemaphoreType.DMA,
    ],
)
def _sc_gather_add(xa_hbm, xbp_hbm, row_hbm, col_hbm, g_hbm,
                   idxr, idxc, bufa, bufb, sema, semb):
    wid = _wid()

    def step(t, _):
        j = wid + t * NW
        pltpu.sync_copy(row_hbm.at[j], idxr)
        pltpu.sync_copy(col_hbm.at[j], idxc)
        cpa = pltpu.async_copy(xa_hbm.at[idxr], bufa, sema)
        cpb = pltpu.async_copy(xbp_hbm.at[idxc], bufb, semb)
        cpa.wait()
        cpb.wait()

        def add_row(r, _):
            for cidx in range(VPL):
                sl = pl.ds(cidx * 16, 16)
                bufa[r, sl] = bufa[r, sl] + bufb[r, sl]
            return 0

        lax.fori_loop(0, EC, add_row, 0)
        pltpu.sync_copy(bufa, g_hbm.at[pl.ds(j * EC, EC)])
        return 0

    lax.fori_loop(0, CPW, step, 0)


@functools.partial(
    pl.kernel,
    out_type=jax.ShapeDtypeStruct((NC, NP, H), jnp.float32),
    mesh=_mesh,
    scratch_types=[
        pltpu.VMEM((CPW, EC), jnp.int32),
        pltpu.VMEM((NBS, EC, H), jnp.float32),
        pltpu.VMEM_SHARED((NP, H), jnp.float32),
    ] + [pltpu.SemaphoreType.DMA] * (2 * NBS),
)
def _sc_scatter_add(ef_hbm, rowp_hbm, aggp_hbm, idxv, ebuf, agg_sh, *sems):
    slm, swm = sems[:NBS], sems[NBS:]
    cid = lax.axis_index("c")
    sid = lax.axis_index("s")
    wid = _wid()
    c0 = wid * CPW
    pltpu.sync_copy(rowp_hbm.at[pl.ds(c0, CPW)], idxv)

    def zero_row(r, _):
        for cidx in range(VPL):
            ebuf[0, r, pl.ds(cidx * 16, 16)] = jnp.zeros((16,), jnp.float32)
        return 0

    lax.fori_loop(0, ZR, zero_row, 0)
    for q in range(ROWS_PER_TILE // ZR):
        pltpu.sync_copy(ebuf.at[0],
                        agg_sh.at[pl.ds(sid * ROWS_PER_TILE + q * ZR, ZR)])
    plsc.subcore_barrier()

    def l_cp(t, b):
        return pltpu.make_async_copy(
            ef_hbm.at[pl.ds((c0 + t) * EC, EC)], ebuf.at[b], slm[b])

    def s_cp(t, b):
        return pltpu.make_async_copy(ebuf.at[b], agg_sh.at[idxv.at[t]], swm[b])

    for b in range(NBS):
        l_cp(b, b).start()

    def group(gi, _):
        base = gi * NBS
        for b in range(NBS):
            t = base + b
            l_cp(t, b).wait()
            pltpu.async_copy(ebuf.at[b], agg_sh.at[idxv.at[t]], swm[b], add=True)
        for b in range(NBS):
            t = base + b
            s_cp(t, b).wait()

            @pl.when(t + NBS < CPW)
            def _():
                l_cp(t + NBS, b).start()
        return 0

    lax.fori_loop(0, NGRP_S, group, 0)
    plsc.subcore_barrier()

    pltpu.sync_copy(agg_sh.at[pl.ds(sid * ROWS_PER_TILE, ROWS_PER_TILE)],
                    aggp_hbm.at[cid, pl.ds(sid * ROWS_PER_TILE, ROWS_PER_TILE)])


def _silu(x):
    return x * jax.nn.sigmoid(x)


def _tc_input_body(h_ref, win, binr, wea, web, ben, xo, xao, xbo):
    x = jnp.dot(h_ref[...], win[...], preferred_element_type=jnp.float32)
    x = x + binr[...]
    xo[...] = x
    xao[...] = jnp.dot(x, wea[...], preferred_element_type=jnp.float32)
    xbo[...] = jnp.dot(x, web[...], preferred_element_type=jnp.float32) + ben[...]


def _tc_edge_body(g_ref, w2, b2, ef_ref):
    t = _silu(g_ref[...])
    u = jnp.dot(t, w2[...], preferred_element_type=jnp.float32) + b2[...]
    rowid = (jax.lax.broadcasted_iota(jnp.int32, (BE, H), 0)
             + pl.program_id(0) * BE)
    ef_ref[...] = jnp.where(rowid < E, _silu(u), 0.0)


def _tc_node_body(x_ref, aggp_ref, wn1a, wn1b, bn1r, wn2, bn2r,
                  wea, web, ben, xo, xao, xbo):
    x = x_ref[...]
    agg = (aggp_ref[0] + aggp_ref[1]) * (1.0 / C)
    t = _silu(jnp.dot(x, wn1a[...], preferred_element_type=jnp.float32)
              + jnp.dot(agg, wn1b[...], preferred_element_type=jnp.float32)
              + bn1r[...])
    xn = x + jnp.dot(t, wn2[...], preferred_element_type=jnp.float32) + bn2r[...]
    xo[...] = xn
    xao[...] = jnp.dot(xn, wea[...], preferred_element_type=jnp.float32)
    xbo[...] = jnp.dot(xn, web[...], preferred_element_type=jnp.float32) + ben[...]


def _tc_node_final_body(x_ref, aggp_ref, wn1a, wn1b, bn1r, wn2, bn2r,
                        wout, boutr, yo):
    x = x_ref[...]
    agg = (aggp_ref[0] + aggp_ref[1]) * (1.0 / C)
    t = _silu(jnp.dot(x, wn1a[...], preferred_element_type=jnp.float32)
              + jnp.dot(agg, wn1b[...], preferred_element_type=jnp.float32)
              + bn1r[...])
    xn = x + jnp.dot(t, wn2[...], preferred_element_type=jnp.float32) + bn2r[...]
    yo[...] = jnp.dot(xn, wout[...], preferred_element_type=jnp.float32) + boutr[...]


BN = 2000
BE = 2048


def _wspec(shape):
    return pl.BlockSpec(shape, lambda i: (0,) * len(shape))


_node_out = [jax.ShapeDtypeStruct((N, H), jnp.float32)] * 3
_nblock = pl.BlockSpec((BN, H), lambda i: (i, 0))
_ablock = pl.BlockSpec((NC, BN, H), lambda i: (0, i, 0))

_tc_input = pl.pallas_call(
    _tc_input_body,
    grid=(N // BN,),
    in_specs=[_nblock, _wspec((D, H)), _wspec((1, H)), _wspec((H, H)),
              _wspec((H, H)), _wspec((1, H))],
    out_specs=[_nblock] * 3,
    out_shape=_node_out,
)

_tc_edge = pl.pallas_call(
    _tc_edge_body,
    grid=(E_PAD // BE,),
    in_specs=[pl.BlockSpec((BE, H), lambda i: (i, 0)), _wspec((H, H)),
              _wspec((1, H))],
    out_specs=pl.BlockSpec((BE, H), lambda i: (i, 0)),
    out_shape=jax.ShapeDtypeStruct((E_PAD, H), jnp.float32),
)

_tc_node = pl.pallas_call(
    _tc_node_body,
    grid=(N // BN,),
    in_specs=[_nblock, _ablock] + [_wspec((H, H)), _wspec((H, H)),
              _wspec((1, H)), _wspec((H, H)), _wspec((1, H)),
              _wspec((H, H)), _wspec((H, H)), _wspec((1, H))],
    out_specs=[_nblock] * 3,
    out_shape=_node_out,
)

_tc_node_final = pl.pallas_call(
    _tc_node_final_body,
    grid=(N // BN,),
    in_specs=[_nblock, _ablock] + [_wspec((H, H)), _wspec((H, H)),
              _wspec((1, H)), _wspec((H, H)), _wspec((1, H)),
              _wspec((H, D)), _wspec((1, D))],
    out_specs=pl.BlockSpec((BN, D), lambda i: (i, 0)),
    out_shape=jax.ShapeDtypeStruct((N, D), jnp.float32),
)


def kernel(h, edges, Win, bin_, We1, be1, We2, be2, Wn1, bn1, Wn2, bn2,
           Wout, bout):
    pad = E_PAD - E
    row_pad = jnp.pad(edges[0], (0, pad)).reshape(NCHUNK_PAD, EC)
    col_pad = jnp.pad(edges[1], (0, pad)).reshape(NCHUNK_PAD, EC)

    b2 = lambda v: v.reshape(1, -1)

    x, xa, xbp = _tc_input(h, Win, b2(bin_), We1[0, :H], We1[0, H:], b2(be1[0]))
    for i in range(N_LAYERS):
        g = _sc_gather_add(xa, xbp, row_pad, col_pad)
        ef = _tc_edge(g, We2[i], b2(be2[i]))
        aggp = _sc_scatter_add(ef, row_pad)
        if i < N_LAYERS - 1:
            x, xa, xbp = _tc_node(x, aggp, Wn1[i, :H], Wn1[i, H:], b2(bn1[i]),
                                  Wn2[i], b2(bn2[i]), We1[i + 1, :H],
                                  We1[i + 1, H:], b2(be1[i + 1]))
        else:
            y = _tc_node_final(x, aggp, Wn1[i, :H], Wn1[i, H:], b2(bn1[i]),
                               Wn2[i], b2(bn2[i]), Wout, b2(bout))
    return y


# speedup vs baseline: 1.3792x; 1.3792x over previous
import functools

import jax
import jax.numpy as jnp
from jax import lax
from jax.experimental import pallas as pl
from jax.experimental.pallas import tpu as pltpu
from jax.experimental.pallas import tpu_sc as plsc

N_LAYERS = 4
C = 1.0
N, E, D, H = 10000, 320000, 128, 128

NC, NS = 2, 16
NW = NC * NS
EC = 128
CPW = 80
NCHUNK_PAD = NW * CPW
E_PAD = NCHUNK_PAD * EC
VPL = H // 16

NBG = 2
NGRP_G = CPW // NBG
NBS = 2
NGRP_S = CPW // NBS

NP = 10240
ROWS_PER_TILE = NP // NS
ZR = 128

_mesh = plsc.VectorSubcoreMesh(core_axis_name="c", subcore_axis_name="s")


def _wid():
    return lax.axis_index("s") * NC + lax.axis_index("c")


@functools.partial(
    pl.kernel,
    out_type=jax.ShapeDtypeStruct((E_PAD, H), jnp.float32),
    mesh=_mesh,
    scratch_types=[
        pltpu.VMEM((EC,), jnp.int32),
        pltpu.VMEM((EC,), jnp.int32),
        pltpu.VMEM((EC, H), jnp.float32),
        pltpu.VMEM((EC, H), jnp.float32),
        pltpu.SemaphoreType.DMA,
        pltpu.SemaphoreType.DMA,
    ],
)
def _sc_gather_add(xa_hbm, xbp_hbm, row_hbm, col_hbm, g_hbm,
                   idxr, idxc, bufa, bufb, sema, semb):
    wid = _wid()

    def step(t, _):
        j = wid + t * NW
        pltpu.sync_copy(row_hbm.at[j], idxr)
        pltpu.sync_copy(col_hbm.at[j], idxc)
        cpa = pltpu.async_copy(xa_hbm.at[idxr], bufa, sema)
        cpb = pltpu.async_copy(xbp_hbm.at[idxc], bufb, semb)
        cpa.wait()
        cpb.wait()

        def add_row(r, _):
            for cidx in range(VPL):
                sl = pl.ds(cidx * 16, 16)
                bufa[r, sl] = bufa[r, sl] + bufb[r, sl]
            return 0

        lax.fori_loop(0, EC, add_row, 0)
        pltpu.sync_copy(bufa, g_hbm.at[pl.ds(j * EC, EC)])
        return 0

    lax.fori_loop(0, CPW, step, 0)


@functools.partial(
    pl.kernel,
    out_type=jax.ShapeDtypeStruct((NC, NP, H), jnp.float32),
    mesh=_mesh,
    scratch_types=[
        pltpu.VMEM((CPW, EC), jnp.int32),
        pltpu.VMEM((NBS, EC, H), jnp.float32),
        pltpu.VMEM_SHARED((NP, H), jnp.float32),
    ] + [pltpu.SemaphoreType.DMA] * (2 * NBS),
)
def _sc_scatter_add(ef_hbm, rowp_hbm, aggp_hbm, idxv, ebuf, agg_sh, *sems):
    slm, swm = sems[:NBS], sems[NBS:]
    cid = lax.axis_index("c")
    sid = lax.axis_index("s")
    wid = _wid()
    c0 = wid * CPW
    pltpu.sync_copy(rowp_hbm.at[pl.ds(c0, CPW)], idxv)

    def zero_row(r, _):
        for cidx in range(VPL):
            ebuf[0, r, pl.ds(cidx * 16, 16)] = jnp.zeros((16,), jnp.float32)
        return 0

    lax.fori_loop(0, ZR, zero_row, 0)
    for q in range(ROWS_PER_TILE // ZR):
        pltpu.sync_copy(ebuf.at[0],
                        agg_sh.at[pl.ds(sid * ROWS_PER_TILE + q * ZR, ZR)])
    plsc.subcore_barrier()

    def l_cp(t, b):
        return pltpu.make_async_copy(
            ef_hbm.at[pl.ds((c0 + t) * EC, EC)], ebuf.at[b], slm[b])

    def s_cp(t, b):
        return pltpu.make_async_copy(ebuf.at[b], agg_sh.at[idxv.at[t]], swm[b])

    for b in range(NBS):
        l_cp(b, b).start()

    def group(gi, _):
        base = gi * NBS
        for b in range(NBS):
            t = base + b
            l_cp(t, b).wait()
            pltpu.async_copy(ebuf.at[b], agg_sh.at[idxv.at[t]], swm[b], add=True)
        for b in range(NBS):
            t = base + b
            s_cp(t, b).wait()

            @pl.when(t + NBS < CPW)
            def _():
                l_cp(t + NBS, b).start()
        return 0

    lax.fori_loop(0, NGRP_S, group, 0)
    plsc.subcore_barrier()

    pltpu.sync_copy(agg_sh.at[pl.ds(sid * ROWS_PER_TILE, ROWS_PER_TILE)],
                    aggp_hbm.at[cid, pl.ds(sid * ROWS_PER_TILE, ROWS_PER_TILE)])


def _silu(x):
    return x * jax.nn.sigmoid(x)


def _tc_input_body(h_ref, win, binr, wea, web, ben, xo, xao, xbo):
    x = jnp.dot(h_ref[...], win[...], preferred_element_type=jnp.float32)
    x = x + binr[...]
    xo[...] = x
    xao[...] = jnp.dot(x, wea[...], preferred_element_type=jnp.float32)
    xbo[...] = jnp.dot(x, web[...], preferred_element_type=jnp.float32) + ben[...]


def _tc_edge_body(g_ref, w2, b2, ef_ref):
    t = _silu(g_ref[...])
    u = jnp.dot(t, w2[...], preferred_element_type=jnp.float32) + b2[...]
    rowid = (jax.lax.broadcasted_iota(jnp.int32, (BE, H), 0)
             + pl.program_id(0) * BE)
    ef_ref[...] = jnp.where(rowid < E, _silu(u), 0.0)


def _tc_node_body(x_ref, aggp_ref, wn1a, wn1b, bn1r, wn2, bn2r,
                  wea, web, ben, xo, xao, xbo):
    x = x_ref[...]
    agg = (aggp_ref[0] + aggp_ref[1]) * (1.0 / C)
    t = _silu(jnp.dot(x, wn1a[...], preferred_element_type=jnp.float32)
              + jnp.dot(agg, wn1b[...], preferred_element_type=jnp.float32)
              + bn1r[...])
    xn = x + jnp.dot(t, wn2[...], preferred_element_type=jnp.float32) + bn2r[...]
    xo[...] = xn
    xao[...] = jnp.dot(xn, wea[...], preferred_element_type=jnp.float32)
    xbo[...] = jnp.dot(xn, web[...], preferred_element_type=jnp.float32) + ben[...]


def _tc_node_final_body(x_ref, aggp_ref, wn1a, wn1b, bn1r, wn2, bn2r,
                        wout, boutr, yo):
    x = x_ref[...]
    agg = (aggp_ref[0] + aggp_ref[1]) * (1.0 / C)
    t = _silu(jnp.dot(x, wn1a[...], preferred_element_type=jnp.float32)
              + jnp.dot(agg, wn1b[...], preferred_element_type=jnp.float32)
              + bn1r[...])
    xn = x + jnp.dot(t, wn2[...], preferred_element_type=jnp.float32) + bn2r[...]
    yo[...] = jnp.dot(xn, wout[...], preferred_element_type=jnp.float32) + boutr[...]


BN = 2000
BE = 2048


def _wspec(shape):
    return pl.BlockSpec(shape, lambda i: (0,) * len(shape))


_node_out = [jax.ShapeDtypeStruct((N, H), jnp.float32)] * 3
_nblock = pl.BlockSpec((BN, H), lambda i: (i, 0))
_ablock = pl.BlockSpec((NC, BN, H), lambda i: (0, i, 0))

_tc_input = pl.pallas_call(
    _tc_input_body,
    grid=(N // BN,),
    in_specs=[_nblock, _wspec((D, H)), _wspec((1, H)), _wspec((H, H)),
              _wspec((H, H)), _wspec((1, H))],
    out_specs=[_nblock] * 3,
    out_shape=_node_out,
)

_tc_edge = pl.pallas_call(
    _tc_edge_body,
    grid=(E_PAD // BE,),
    in_specs=[pl.BlockSpec((BE, H), lambda i: (i, 0)), _wspec((H, H)),
              _wspec((1, H))],
    out_specs=pl.BlockSpec((BE, H), lambda i: (i, 0)),
    out_shape=jax.ShapeDtypeStruct((E_PAD, H), jnp.float32),
)

_tc_node = pl.pallas_call(
    _tc_node_body,
    grid=(N // BN,),
    in_specs=[_nblock, _ablock] + [_wspec((H, H)), _wspec((H, H)),
              _wspec((1, H)), _wspec((H, H)), _wspec((1, H)),
              _wspec((H, H)), _wspec((H, H)), _wspec((1, H))],
    out_specs=[_nblock] * 3,
    out_shape=_node_out,
)

_tc_node_final = pl.pallas_call(
    _tc_node_final_body,
    grid=(N // BN,),
    in_specs=[_nblock, _ablock] + [_wspec((H, H)), _wspec((H, H)),
              _wspec((1, H)), _wspec((H, H)), _wspec((1, H)),
              _wspec((H, D)), _wspec((1, D))],
    out_specs=pl.BlockSpec((BN, D), lambda i: (i, 0)),
    out_shape=jax.ShapeDtypeStruct((N, D), jnp.float32),
)


def kernel(h, edges, Win, bin_, We1, be1, We2, be2, Wn1, bn1, Wn2, bn2,
           Wout, bout):
    pad = E_PAD - E
    fill = jnp.arange(pad, dtype=jnp.int32) % N
    row_pad = jnp.concatenate([edges[0], fill]).reshape(NCHUNK_PAD, EC)
    col_pad = jnp.concatenate([edges[1], fill]).reshape(NCHUNK_PAD, EC)

    b2 = lambda v: v.reshape(1, -1)

    x, xa, xbp = _tc_input(h, Win, b2(bin_), We1[0, :H], We1[0, H:], b2(be1[0]))
    for i in range(N_LAYERS):
        g = _sc_gather_add(xa, xbp, row_pad, col_pad)
        ef = _tc_edge(g, We2[i], b2(be2[i]))
        aggp = _sc_scatter_add(ef, row_pad)
        if i < N_LAYERS - 1:
            x, xa, xbp = _tc_node(x, aggp, Wn1[i, :H], Wn1[i, H:], b2(bn1[i]),
                                  Wn2[i], b2(bn2[i]), We1[i + 1, :H],
                                  We1[i + 1, H:], b2(be1[i + 1]))
        else:
            y = _tc_node_final(x, aggp, Wn1[i, :H], Wn1[i, H:], b2(bn1[i]),
                               Wn2[i], b2(bn2[i]), Wout, b2(bout))
    return y


# speedup vs baseline: 1.9196x; 1.3919x over previous
import functools

import jax
import jax.numpy as jnp
from jax import lax
from jax.experimental import pallas as pl
from jax.experimental.pallas import tpu as pltpu
from jax.experimental.pallas import tpu_sc as plsc

N_LAYERS = 4
C = 1.0
N, E, D, H = 10000, 320000, 128, 128

NC, NS = 2, 16
NW = NC * NS
EC = 128
CPW = 80
NCHUNK_PAD = NW * CPW
E_PAD = NCHUNK_PAD * EC
VPL = H // 16

NBG = 2
NGRP_G = CPW // NBG
NBS = 2
NGRP_S = CPW // NBS

NP = 10240
ROWS_PER_TILE = NP // NS
ZR = 128

_mesh = plsc.VectorSubcoreMesh(core_axis_name="c", subcore_axis_name="s")


def _wid():
    return lax.axis_index("s") * NC + lax.axis_index("c")


NIX = 4


@functools.partial(
    pl.kernel,
    out_type=jax.ShapeDtypeStruct((E_PAD, H), jnp.float32),
    mesh=_mesh,
    scratch_types=(
        [pltpu.VMEM((EC,), jnp.int32)] * (2 * NIX)
        + [pltpu.VMEM((EC, H), jnp.float32)] * (3 * NBG)
        + [pltpu.SemaphoreType.DMA] * (NIX + 3 * NBG)
    ),
)
def _sc_gather_add(xa_hbm, xbp_hbm, row_hbm, col_hbm, g_hbm, *refs):
    ixr, ixc = list(refs[:NIX]), list(refs[NIX:2 * NIX])
    k = 2 * NIX
    bas = list(refs[k:k + NBG])
    bbs = list(refs[k + NBG:k + 2 * NBG])
    bos = list(refs[k + 2 * NBG:k + 3 * NBG])
    sems = refs[k + 3 * NBG:]
    si = sems[:NIX]
    sga = sems[NIX:NIX + NBG]
    sgb = sems[NIX + NBG:NIX + 2 * NBG]
    sw = sems[NIX + 2 * NBG:]
    wid = _wid()

    def ixr_cp(t, q):
        return pltpu.make_async_copy(row_hbm.at[wid + t * NW], ixr[q], si[q])

    def ixc_cp(t, q):
        return pltpu.make_async_copy(col_hbm.at[wid + t * NW], ixc[q], si[q])

    def ga_cp(q, b):
        return pltpu.make_async_copy(xa_hbm.at[ixr[q]], bas[b], sga[b])

    def gb_cp(q, b):
        return pltpu.make_async_copy(xbp_hbm.at[ixc[q]], bbs[b], sgb[b])

    def w_cp(t, b):
        return pltpu.make_async_copy(
            bos[b], g_hbm.at[pl.ds((wid + t * NW) * EC, EC)], sw[b])

    for q in range(NIX):
        ixr_cp(q, q).start()
        ixc_cp(q, q).start()
    for t0 in range(NBG):
        ixr_cp(t0, t0).wait()
        ixc_cp(t0, t0).wait()
        ga_cp(t0, t0).start()
        gb_cp(t0, t0).start()

    def group(gi, _):
        base = gi * NIX
        for lane in range(NIX):
            t = base + lane
            b = lane % NBG
            ga_cp(lane, b).wait()
            gb_cp(lane, b).wait()

            @pl.when(t >= NBG)
            def _():
                w_cp(t - NBG, b).wait()

            def add_row(r, _):
                for cidx in range(VPL):
                    sl = pl.ds(cidx * 16, 16)
                    bos[b][r, sl] = bas[b][r, sl] + bbs[b][r, sl]
                return 0

            lax.fori_loop(0, EC, add_row, 0)

            @pl.when(t + NIX < CPW)
            def _():
                ixr_cp(t + NIX, lane).start()
                ixc_cp(t + NIX, lane).start()

            @pl.when(t + NBG < CPW)
            def _():
                qn = (lane + NBG) % NIX
                ixr_cp(t + NBG, qn).wait()
                ixc_cp(t + NBG, qn).wait()
                ga_cp(qn, b).start()
                gb_cp(qn, b).start()

            w_cp(t, b).start()
        return 0

    lax.fori_loop(0, CPW // NIX, group, 0)
    for b in range(NBG):
        w_cp(CPW - NBG + b, b).wait()


@functools.partial(
    pl.kernel,
    out_type=jax.ShapeDtypeStruct((NC, NP, H), jnp.float32),
    mesh=_mesh,
    scratch_types=[
        pltpu.VMEM((CPW, EC), jnp.int32),
        pltpu.VMEM((NBS, EC, H), jnp.float32),
        pltpu.VMEM_SHARED((NP, H), jnp.float32),
    ] + [pltpu.SemaphoreType.DMA] * (2 * NBS),
)
def _sc_scatter_add(ef_hbm, rowp_hbm, aggp_hbm, idxv, ebuf, agg_sh, *sems):
    slm, swm = sems[:NBS], sems[NBS:]
    cid = lax.axis_index("c")
    sid = lax.axis_index("s")
    wid = _wid()
    c0 = wid * CPW
    pltpu.sync_copy(rowp_hbm.at[pl.ds(c0, CPW)], idxv)

    def zero_row(r, _):
        for cidx in range(VPL):
            ebuf[0, r, pl.ds(cidx * 16, 16)] = jnp.zeros((16,), jnp.float32)
        return 0

    lax.fori_loop(0, ZR, zero_row, 0)
    for q in range(ROWS_PER_TILE // ZR):
        pltpu.sync_copy(ebuf.at[0],
                        agg_sh.at[pl.ds(sid * ROWS_PER_TILE + q * ZR, ZR)])
    plsc.subcore_barrier()

    def l_cp(t, b):
        return pltpu.make_async_copy(
            ef_hbm.at[pl.ds((c0 + t) * EC, EC)], ebuf.at[b], slm[b])

    def s_cp(t, b):
        return pltpu.make_async_copy(ebuf.at[b], agg_sh.at[idxv.at[t]], swm[b])

    for b in range(NBS):
        l_cp(b, b).start()

    def group(gi, _):
        base = gi * NBS
        for b in range(NBS):
            t = base + b
            l_cp(t, b).wait()
            pltpu.async_copy(ebuf.at[b], agg_sh.at[idxv.at[t]], swm[b], add=True)
        for b in range(NBS):
            t = base + b
            s_cp(t, b).wait()

            @pl.when(t + NBS < CPW)
            def _():
                l_cp(t + NBS, b).start()
        return 0

    lax.fori_loop(0, NGRP_S, group, 0)
    plsc.subcore_barrier()

    pltpu.sync_copy(agg_sh.at[pl.ds(sid * ROWS_PER_TILE, ROWS_PER_TILE)],
                    aggp_hbm.at[cid, pl.ds(sid * ROWS_PER_TILE, ROWS_PER_TILE)])


def _silu(x):
    return x * jax.nn.sigmoid(x)


def _tc_input_body(h_ref, win, binr, wea, web, ben, xo, xao, xbo):
    x = jnp.dot(h_ref[...], win[...], preferred_element_type=jnp.float32)
    x = x + binr[...]
    xo[...] = x
    xao[...] = jnp.dot(x, wea[...], preferred_element_type=jnp.float32)
    xbo[...] = jnp.dot(x, web[...], preferred_element_type=jnp.float32) + ben[...]


def _tc_edge_body(g_ref, w2, b2, ef_ref):
    t = _silu(g_ref[...])
    u = jnp.dot(t, w2[...], preferred_element_type=jnp.float32) + b2[...]
    rowid = (jax.lax.broadcasted_iota(jnp.int32, (BE, H), 0)
             + pl.program_id(0) * BE)
    ef_ref[...] = jnp.where(rowid < E, _silu(u), 0.0)


def _tc_node_body(x_ref, aggp_ref, wn1a, wn1b, bn1r, wn2, bn2r,
                  wea, web, ben, xo, xao, xbo):
    x = x_ref[...]
    agg = (aggp_ref[0] + aggp_ref[1]) * (1.0 / C)
    t = _silu(jnp.dot(x, wn1a[...], preferred_element_type=jnp.float32)
              + jnp.dot(agg, wn1b[...], preferred_element_type=jnp.float32)
              + bn1r[...])
    xn = x + jnp.dot(t, wn2[...], preferred_element_type=jnp.float32) + bn2r[...]
    xo[...] = xn
    xao[...] = jnp.dot(xn, wea[...], preferred_element_type=jnp.float32)
    xbo[...] = jnp.dot(xn, web[...], preferred_element_type=jnp.float32) + ben[...]


def _tc_node_final_body(x_ref, aggp_ref, wn1a, wn1b, bn1r, wn2, bn2r,
                        wout, boutr, yo):
    x = x_ref[...]
    agg = (aggp_ref[0] + aggp_ref[1]) * (1.0 / C)
    t = _silu(jnp.dot(x, wn1a[...], preferred_element_type=jnp.float32)
              + jnp.dot(agg, wn1b[...], preferred_element_type=jnp.float32)
              + bn1r[...])
    xn = x + jnp.dot(t, wn2[...], preferred_element_type=jnp.float32) + bn2r[...]
    yo[...] = jnp.dot(xn, wout[...], preferred_element_type=jnp.float32) + boutr[...]


BN = 2000
BE = 2048


def _wspec(shape):
    return pl.BlockSpec(shape, lambda i: (0,) * len(shape))


_node_out = [jax.ShapeDtypeStruct((N, H), jnp.float32)] * 3
_nblock = pl.BlockSpec((BN, H), lambda i: (i, 0))
_ablock = pl.BlockSpec((NC, BN, H), lambda i: (0, i, 0))

_tc_input = pl.pallas_call(
    _tc_input_body,
    grid=(N // BN,),
    in_specs=[_nblock, _wspec((D, H)), _wspec((1, H)), _wspec((H, H)),
              _wspec((H, H)), _wspec((1, H))],
    out_specs=[_nblock] * 3,
    out_shape=_node_out,
)

_tc_edge = pl.pallas_call(
    _tc_edge_body,
    grid=(E_PAD // BE,),
    in_specs=[pl.BlockSpec((BE, H), lambda i: (i, 0)), _wspec((H, H)),
              _wspec((1, H))],
    out_specs=pl.BlockSpec((BE, H), lambda i: (i, 0)),
    out_shape=jax.ShapeDtypeStruct((E_PAD, H), jnp.float32),
)

_tc_node = pl.pallas_call(
    _tc_node_body,
    grid=(N // BN,),
    in_specs=[_nblock, _ablock] + [_wspec((H, H)), _wspec((H, H)),
              _wspec((1, H)), _wspec((H, H)), _wspec((1, H)),
              _wspec((H, H)), _wspec((H, H)), _wspec((1, H))],
    out_specs=[_nblock] * 3,
    out_shape=_node_out,
)

_tc_node_final = pl.pallas_call(
    _tc_node_final_body,
    grid=(N // BN,),
    in_specs=[_nblock, _ablock] + [_wspec((H, H)), _wspec((H, H)),
              _wspec((1, H)), _wspec((H, H)), _wspec((1, H)),
              _wspec((H, D)), _wspec((1, D))],
    out_specs=pl.BlockSpec((BN, D), lambda i: (i, 0)),
    out_shape=jax.ShapeDtypeStruct((N, D), jnp.float32),
)


def kernel(h, edges, Win, bin_, We1, be1, We2, be2, Wn1, bn1, Wn2, bn2,
           Wout, bout):
    pad = E_PAD - E
    fill = jnp.arange(pad, dtype=jnp.int32) % N
    row_pad = jnp.concatenate([edges[0], fill]).reshape(NCHUNK_PAD, EC)
    col_pad = jnp.concatenate([edges[1], fill]).reshape(NCHUNK_PAD, EC)

    b2 = lambda v: v.reshape(1, -1)

    x, xa, xbp = _tc_input(h, Win, b2(bin_), We1[0, :H], We1[0, H:], b2(be1[0]))
    for i in range(N_LAYERS):
        g = _sc_gather_add(xa, xbp, row_pad, col_pad)
        ef = _tc_edge(g, We2[i], b2(be2[i]))
        aggp = _sc_scatter_add(ef, row_pad)
        if i < N_LAYERS - 1:
            x, xa, xbp = _tc_node(x, aggp, Wn1[i, :H], Wn1[i, H:], b2(bn1[i]),
                                  Wn2[i], b2(bn2[i]), We1[i + 1, :H],
                                  We1[i + 1, H:], b2(be1[i + 1]))
        else:
            y = _tc_node_final(x, aggp, Wn1[i, :H], Wn1[i, H:], b2(bn1[i]),
                               Wn2[i], b2(bn2[i]), Wout, b2(bout))
    return y
